# Initial kernel scaffold; baseline (speedup 1.0000x reference)
import jax
import jax.numpy as jnp
from jax import lax
from jax.experimental import pallas as pl
from jax.experimental.pallas import tpu as pltpu
from jax.experimental.pallas import tpu_sc as plsc

F32 = jnp.float32
I32 = jnp.int32

N = 10000
NPAD = 10240
NSLICE = NPAD // 16
E = 160000
ET = E + N
ETP = 170496
EW = ETP // 32
NCH = EW // 16
ELP = 160256
EWL = ELP // 32
NCHL = EWL // 16

H1 = 160
H2 = 112


def _mesh():
    return plsc.VectorSubcoreMesh(core_axis_name="c", subcore_axis_name="s")


def _worker_id():
    return lax.axis_index("c") * 16 + lax.axis_index("s")


def _zero_1d(ref, nwords):
    zero = jnp.zeros((16,), F32)

    def zb(k, carry):
        ref[pl.ds(k * 16, 16)] = zero
        return carry

    lax.fori_loop(0, nwords // 16, zb, None)


def _combine_to_hbm(local_ref, shs, tmp_v, red_v, out_hbm):
    c = lax.axis_index("c")
    sb = lax.axis_index("s")
    pltpu.sync_copy(local_ref, shs.at[sb])
    plsc.subcore_barrier()
    off = sb * NSLICE
    pltpu.sync_copy(shs.at[0, pl.ds(off, NSLICE)], red_v)
    for r in range(1, 16):
        pltpu.sync_copy(shs.at[r, pl.ds(off, NSLICE)], tmp_v)

        def addk(k, carry):
            red_v[pl.ds(k * 16, 16)] = (
                red_v[pl.ds(k * 16, 16)] + tmp_v[pl.ds(k * 16, 16)]
            )
            return carry

        lax.fori_loop(0, NSLICE // 16, addk, None)
    pltpu.sync_copy(red_v, out_hbm.at[c, pl.ds(off, NSLICE)])


def _gat_stage1(src_p, dst_p, att_p, tables, H):
    npairs = len(tables)
    NH = npairs * H
    nj = H // 16

    def body(src_hbm, dst_hbm, att_hbm, *rest):
        tabs = rest[:2 * npairs]
        ex_hbm, spart_hbm = rest[2 * npairs], rest[2 * npairs + 1]
        scr = rest[2 * npairs + 2:]
        att_v, src_v, dst_v = scr[0], scr[1], scr[2]
        rows = scr[3:3 + 2 * npairs]
        P, ex_v, s_loc, tmp_v, red_v, shs, sem = scr[3 + 2 * npairs:]

        wid = _worker_id()
        pltpu.sync_copy(att_hbm, att_v)
        _zero_1d(s_loc, NPAD)
        att_regs = [att_v[pl.ds(k * 16, 16)] for k in range(NH // 16)]
        iot = lax.iota(I32, 16)

        def chunk(i, carry):
            base = wid * EW + i * 16
            pltpu.sync_copy(src_hbm.at[pl.ds(base, 16)], src_v)
            pltpu.sync_copy(dst_hbm.at[pl.ds(base, 16)], dst_v)
            cps = []
            for p in range(npairs):
                cps.append(
                    pltpu.async_copy(tabs[2 * p].at[src_v], rows[2 * p], sem))
                cps.append(
                    pltpu.async_copy(tabs[2 * p + 1].at[dst_v], rows[2 * p + 1],
                                     sem))
            for cp in cps:
                cp.wait()

            def edge(e, ecarry):
                acc = jnp.zeros((16,), F32)
                for p in range(npairs):
                    for j in range(nj):
                        v = (rows[2 * p][e, pl.ds(j * 16, 16)]
                             + rows[2 * p + 1][e, pl.ds(j * 16, 16)])
                        v = jnp.where(v >= 0, v, 0.2 * v)
                        acc = acc + v * att_regs[p * nj + j]
                P[e, :] = acc
                return ecarry

            lax.fori_loop(0, 16, edge, None)
            t = jnp.zeros((16,), F32)
            for col in range(16):
                t = t + plsc.load_gather(P, [iot, jnp.full((16,), col, I32)])
            mask = (base + iot) < ET
            exv = jnp.where(mask, jnp.exp(t), 0.0)
            ex_v[...] = exv
            pltpu.sync_copy(ex_v, ex_hbm.at[pl.ds(base, 16)])
            plsc.addupdate_scatter(s_loc, [dst_v[...]], exv)
            return carry

        lax.fori_loop(0, NCH, chunk, None)
        _combine_to_hbm(s_loc, shs, tmp_v, red_v, spart_hbm)

    scratch = (
        [pltpu.VMEM((NH,), F32), pltpu.VMEM((16,), I32), pltpu.VMEM((16,), I32)]
        + [pltpu.VMEM((16, H), F32)] * (2 * npairs)
        + [pltpu.VMEM((16, 16), F32), pltpu.VMEM((16,), F32),
           pltpu.VMEM((NPAD,), F32), pltpu.VMEM((NSLICE,), F32),
           pltpu.VMEM((NSLICE,), F32),
           pltpu.VMEM_SHARED((16, NPAD), F32), pltpu.SemaphoreType.DMA]
    )
    flat_tabs = [a for pair in tables for a in pair]
    fn = pl.kernel(
        body,
        out_type=[jax.ShapeDtypeStruct((ETP,), F32),
                  jax.ShapeDtypeStruct((2, NPAD), F32)],
        mesh=_mesh(),
        scratch_types=scratch,
    )
    return fn(src_p, dst_p, att_p, *flat_tabs)


def _gat_stage2(src_p, dst_p, ex, spart, xl, H):
    nj = H // 16

    def body(src_hbm, dst_hbm, ex_hbm, sp_hbm, xl_hbm, op_hbm,
             s_tot, tmp_big, src_v, dst_v, ex_v, rows, a_buf, zrow, osh, sem):
        c = lax.axis_index("c")
        sb = lax.axis_index("s")
        wid = _worker_id()
        pltpu.sync_copy(sp_hbm.at[0], s_tot)
        pltpu.sync_copy(sp_hbm.at[1], tmp_big)

        def sk(k, carry):
            s_tot[pl.ds(k * 16, 16)] = (
                s_tot[pl.ds(k * 16, 16)] + tmp_big[pl.ds(k * 16, 16)] + 1e-16
            )
            return carry

        lax.fori_loop(0, NPAD // 16, sk, None)

        zero = jnp.zeros((16,), F32)
        for r in range(16):
            for j in range(nj):
                zrow[r, pl.ds(j * 16, 16)] = zero
        for r in range(NSLICE // 16):
            pltpu.sync_copy(zrow, osh.at[pl.ds(sb * NSLICE + r * 16, 16)])
        plsc.subcore_barrier()

        def chunk(i, carry):
            base = wid * EW + i * 16
            pltpu.sync_copy(src_hbm.at[pl.ds(base, 16)], src_v)
            pltpu.sync_copy(dst_hbm.at[pl.ds(base, 16)], dst_v)
            pltpu.sync_copy(ex_hbm.at[pl.ds(base, 16)], ex_v)
            pltpu.async_copy(xl_hbm.at[src_v], rows, sem).wait()
            sv = plsc.load_gather(s_tot, [dst_v[...]])
            a_buf[...] = ex_v[...] / sv

            def edge(e, ecarry):
                av = plsc.load_gather(a_buf, [jnp.full((16,), 0, I32) + e])
                for j in range(nj):
                    rows[e, pl.ds(j * 16, 16)] = rows[e, pl.ds(j * 16, 16)] * av
                return ecarry

            lax.fori_loop(0, 16, edge, None)
            pltpu.sync_copy(rows, osh.at[dst_v], add=True)
            return carry

        lax.fori_loop(0, NCH, chunk, None)
        plsc.subcore_barrier()
        pltpu.sync_copy(osh.at[pl.ds(sb * NSLICE, NSLICE)],
                        op_hbm.at[c, pl.ds(sb * NSLICE, NSLICE)])

    scratch = [
        pltpu.VMEM((NPAD,), F32), pltpu.VMEM((NPAD,), F32),
        pltpu.VMEM((16,), I32), pltpu.VMEM((16,), I32), pltpu.VMEM((16,), F32),
        pltpu.VMEM((16, H), F32), pltpu.VMEM((16,), F32),
        pltpu.VMEM((16, H), F32),
        pltpu.VMEM_SHARED((NPAD, H), F32), pltpu.SemaphoreType.DMA,
    ]
    fn = pl.kernel(
        body,
        out_type=jax.ShapeDtypeStruct((2, NPAD, H), F32),
        mesh=_mesh(),
        scratch_types=scratch,
    )
    return fn(src_p, dst_p, ex, spart, xl)


def _l3_stage1(src_p, dst_p, xl3, xr3, att3b):
    def body(src_hbm, dst_hbm, xl_hbm, xr_hbm, att_hbm, ex_hbm, spart_hbm,
             xl_v, xr_v, att_v, src_v, dst_v, ex_v, s_loc, tmp_v, red_v, shs,
             sem):
        wid = _worker_id()
        pltpu.sync_copy(xl_hbm, xl_v)
        pltpu.sync_copy(xr_hbm, xr_v)
        pltpu.sync_copy(att_hbm, att_v)
        _zero_1d(s_loc, NPAD)
        att_reg = att_v[...]
        iot = lax.iota(I32, 16)

        def chunk(i, carry):
            base = wid * EW + i * 16
            pltpu.sync_copy(src_hbm.at[pl.ds(base, 16)], src_v)
            pltpu.sync_copy(dst_hbm.at[pl.ds(base, 16)], dst_v)
            sreg = src_v[...]
            dreg = dst_v[...]
            v = plsc.load_gather(xl_v, [sreg]) + plsc.load_gather(xr_v, [dreg])
            t = jnp.where(v >= 0, v, 0.2 * v) * att_reg
            mask = (base + iot) < ET
            exv = jnp.where(mask, jnp.exp(t), 0.0)
            ex_v[...] = exv
            pltpu.sync_copy(ex_v, ex_hbm.at[pl.ds(base, 16)])
            plsc.addupdate_scatter(s_loc, [dreg], exv)
            return carry

        lax.fori_loop(0, NCH, chunk, None)
        _combine_to_hbm(s_loc, shs, tmp_v, red_v, spart_hbm)

    scratch = [
        pltpu.VMEM((NPAD,), F32), pltpu.VMEM((NPAD,), F32),
        pltpu.VMEM((16,), F32),
        pltpu.VMEM((16,), I32), pltpu.VMEM((16,), I32), pltpu.VMEM((16,), F32),
        pltpu.VMEM((NPAD,), F32), pltpu.VMEM((NSLICE,), F32),
        pltpu.VMEM((NSLICE,), F32),
        pltpu.VMEM_SHARED((16, NPAD), F32), pltpu.SemaphoreType.DMA,
    ]
    fn = pl.kernel(
        body,
        out_type=[jax.ShapeDtypeStruct((ETP,), F32),
                  jax.ShapeDtypeStruct((2, NPAD), F32)],
        mesh=_mesh(),
        scratch_types=scratch,
    )
    return fn(src_p, dst_p, xl3, xr3, att3b)


def _l3_stage2(src_p, dst_p, ex3, spart, xl3):
    def body(src_hbm, dst_hbm, ex_hbm, sp_hbm, xl_hbm, op_hbm,
             xl_v, s_tot, tmp_big, src_v, dst_v, ex_v, o_loc, tmp_v, red_v,
             shs, sem):
        wid = _worker_id()
        pltpu.sync_copy(xl_hbm, xl_v)
        pltpu.sync_copy(sp_hbm.at[0], s_tot)
        pltpu.sync_copy(sp_hbm.at[1], tmp_big)

        def sk(k, carry):
            s_tot[pl.ds(k * 16, 16)] = (
                s_tot[pl.ds(k * 16, 16)] + tmp_big[pl.ds(k * 16, 16)] + 1e-16
            )
            return carry

        lax.fori_loop(0, NPAD // 16, sk, None)
        _zero_1d(o_loc, NPAD)

        def chunk(i, carry):
            base = wid * EW + i * 16
            pltpu.sync_copy(src_hbm.at[pl.ds(base, 16)], src_v)
            pltpu.sync_copy(dst_hbm.at[pl.ds(base, 16)], dst_v)
            pltpu.sync_copy(ex_hbm.at[pl.ds(base, 16)], ex_v)
            sreg = src_v[...]
            dreg = dst_v[...]
            alpha = ex_v[...] / plsc.load_gather(s_tot, [dreg])
            o = alpha * plsc.load_gather(xl_v, [sreg])
            plsc.addupdate_scatter(o_loc, [dreg], o)
            return carry

        lax.fori_loop(0, NCH, chunk, None)
        _combine_to_hbm(o_loc, shs, tmp_v, red_v, op_hbm)

    scratch = [
        pltpu.VMEM((NPAD,), F32), pltpu.VMEM((NPAD,), F32),
        pltpu.VMEM((NPAD,), F32),
        pltpu.VMEM((16,), I32), pltpu.VMEM((16,), I32), pltpu.VMEM((16,), F32),
        pltpu.VMEM((NPAD,), F32), pltpu.VMEM((NSLICE,), F32),
        pltpu.VMEM((NSLICE,), F32),
        pltpu.VMEM_SHARED((16, NPAD), F32), pltpu.SemaphoreType.DMA,
    ]
    fn = pl.kernel(
        body,
        out_type=jax.ShapeDtypeStruct((2, NPAD), F32),
        mesh=_mesh(),
        scratch_types=scratch,
    )
    return fn(src_p, dst_p, ex3, spart, xl3)


def _loss_dots(pa, pb, na, nb, z):
    nj = H2 // 16

    def body(pa_hbm, pb_hbm, na_hbm, nb_hbm, z_hbm, dp_hbm, dn_hbm,
             ia, ib, za, zb, P, d_v, sem):
        wid = _worker_id()
        iot = lax.iota(I32, 16)
        for (a_hbm, b_hbm, o_hbm) in ((pa_hbm, pb_hbm, dp_hbm),
                                      (na_hbm, nb_hbm, dn_hbm)):
            def chunk(i, carry, a_hbm=a_hbm, b_hbm=b_hbm, o_hbm=o_hbm):
                base = wid * EWL + i * 16
                pltpu.sync_copy(a_hbm.at[pl.ds(base, 16)], ia)
                pltpu.sync_copy(b_hbm.at[pl.ds(base, 16)], ib)
                cp1 = pltpu.async_copy(z_hbm.at[ia], za, sem)
                cp2 = pltpu.async_copy(z_hbm.at[ib], zb, sem)
                cp1.wait()
                cp2.wait()

                def edge(e, ecarry):
                    acc = jnp.zeros((16,), F32)
                    for j in range(nj):
                        acc = (acc
                               + za[e, pl.ds(j * 16, 16)]
                               * zb[e, pl.ds(j * 16, 16)])
                    P[e, :] = acc
                    return ecarry

                lax.fori_loop(0, 16, edge, None)
                t = jnp.zeros((16,), F32)
                for col in range(16):
                    t = t + plsc.load_gather(P, [iot, jnp.full((16,), col, I32)])
                d_v[...] = t
                pltpu.sync_copy(d_v, o_hbm.at[pl.ds(base, 16)])
                return carry

            lax.fori_loop(0, NCHL, chunk, None)

    scratch = [
        pltpu.VMEM((16,), I32), pltpu.VMEM((16,), I32),
        pltpu.VMEM((16, H2), F32), pltpu.VMEM((16, H2), F32),
        pltpu.VMEM((16, 16), F32), pltpu.VMEM((16,), F32),
        pltpu.SemaphoreType.DMA,
    ]
    fn = pl.kernel(
        body,
        out_type=[jax.ShapeDtypeStruct((ELP,), F32),
                  jax.ShapeDtypeStruct((ELP,), F32)],
        mesh=_mesh(),
        scratch_types=scratch,
    )
    return fn(pa, pb, na, nb, z)


_RB = 2000


def _t1(x, wl0, wl1, wr0, wr1):
    def body(x_ref, w0, w1, w2, w3, o0, o1, o2, o3):
        xb = x_ref[...]
        o0[...] = jnp.dot(xb, w0[...], preferred_element_type=F32)
        o1[...] = jnp.dot(xb, w1[...], preferred_element_type=F32)
        o2[...] = jnp.dot(xb, w2[...], preferred_element_type=F32)
        o3[...] = jnp.dot(xb, w3[...], preferred_element_type=F32)

    return pl.pallas_call(
        body,
        grid=(N // _RB,),
        in_specs=[pl.BlockSpec((_RB, 128), lambda i: (i, 0))]
        + [pl.BlockSpec((128, H1), lambda i: (0, 0))] * 4,
        out_specs=[pl.BlockSpec((_RB, H1), lambda i: (i, 0))] * 4,
        out_shape=[jax.ShapeDtypeStruct((N, H1), F32)] * 4,
    )(x, wl0, wl1, wr0, wr1)


def _t2(o1a, o1b, b1p, wl2p, wr2p):
    def body(oa_ref, ob_ref, b1_ref, wl_ref, wr_ref, xl2_ref, xr2_ref):
        h0 = oa_ref[0] + oa_ref[1]
        h1 = ob_ref[0] + ob_ref[1]
        h = jnp.concatenate([h0, h1], axis=1) + b1_ref[...]
        h = jnp.maximum(h, 0.0)
        xl2_ref[...] = jnp.dot(h, wl_ref[...], preferred_element_type=F32)
        xr2_ref[...] = jnp.dot(h, wr_ref[...], preferred_element_type=F32)

    return pl.pallas_call(
        body,
        grid=(N // _RB,),
        in_specs=[
            pl.BlockSpec((2, _RB, H1), lambda i: (0, i, 0)),
            pl.BlockSpec((2, _RB, H1), lambda i: (0, i, 0)),
            pl.BlockSpec((1, 2 * H1), lambda i: (0, 0)),
            pl.BlockSpec((2 * H1, H2), lambda i: (0, 0)),
            pl.BlockSpec((2 * H1, H2), lambda i: (0, 0)),
        ],
        out_specs=[pl.BlockSpec((_RB, H2), lambda i: (i, 0))] * 2,
        out_shape=[jax.ShapeDtypeStruct((N, H2), F32)] * 2,
    )(o1a, o1b, b1p, wl2p, wr2p)


def _t3(o2, b2p, x, wlin1p, blin1p, wlin2p, blin2p, w3p):
    def body(o2_ref, b2_ref, x_ref, w1_ref, bl1_ref, w2_ref, bl2_ref, w3_ref,
             z_ref, o3c_ref):
        x1 = jnp.maximum(o2_ref[0] + o2_ref[1] + b2_ref[...], 0.0)
        xb = x_ref[...]
        t1 = jnp.maximum(
            jnp.dot(xb, w1_ref[...], preferred_element_type=F32)
            + bl1_ref[...], 0.0)
        xs = x1 + t1
        t2 = jnp.maximum(
            jnp.dot(xb, w2_ref[...], preferred_element_type=F32)
            + bl2_ref[...], 0.0)
        z_ref[...] = x1 + t2
        o3c_ref[...] = jnp.dot(xs, w3_ref[...], preferred_element_type=F32)

    return pl.pallas_call(
        body,
        grid=(N // _RB,),
        in_specs=[
            pl.BlockSpec((2, _RB, H2), lambda i: (0, i, 0)),
            pl.BlockSpec((1, H2), lambda i: (0, 0)),
            pl.BlockSpec((_RB, 128), lambda i: (i, 0)),
            pl.BlockSpec((128, H2), lambda i: (0, 0)),
            pl.BlockSpec((1, H2), lambda i: (0, 0)),
            pl.BlockSpec((128, H2), lambda i: (0, 0)),
            pl.BlockSpec((1, H2), lambda i: (0, 0)),
            pl.BlockSpec((H2, 128), lambda i: (0, 0)),
        ],
        out_specs=[
            pl.BlockSpec((_RB, H2), lambda i: (i, 0)),
            pl.BlockSpec((_RB, 128), lambda i: (i, 0)),
        ],
        out_shape=[
            jax.ShapeDtypeStruct((N, H2), F32),
            jax.ShapeDtypeStruct((N, 128), F32),
        ],
    )(o2, b2p, x, wlin1p, blin1p, wlin2p, blin2p, w3p)


def _t4(dp2, dn2, p3, b3r):
    def body(dp_ref, dn_ref, p3_ref, b3_ref, rl_ref, o3_ref):
        p = jax.nn.sigmoid(dp_ref[...])
        pls = -jnp.mean(jnp.log(p + 1e-15))
        q = jax.nn.sigmoid(dn_ref[...])
        nls = -jnp.mean(jnp.log(1.0 - q + 1e-15))
        rl_ref[0, 0] = pls + nls
        o3_ref[...] = p3_ref[0] + p3_ref[1] + b3_ref[0, 0]

    return pl.pallas_call(
        body,
        out_shape=[jax.ShapeDtypeStruct((1, 1), F32),
                   jax.ShapeDtypeStruct((NPAD,), F32)],
    )(dp2, dn2, p3, b3r)


def kernel(x, edge_index, neg_edge_index, Wl1, Wr1, att1, b1, Wl2, Wr2, att2,
           b2, Wl3, Wr3, att3, b3, Wlin1, blin1, Wlin2, blin2, c1, c2):
    loop = jnp.arange(N, dtype=edge_index.dtype)
    src = jnp.concatenate([edge_index[0], loop])
    dst = jnp.concatenate([edge_index[1], loop])
    src_p = jnp.pad(src, (0, ETP - ET))
    dst_p = jnp.pad(dst, (0, ETP - ET))

    wl1p = jnp.pad(Wl1, ((0, 0), (0, 20)))
    wr1p = jnp.pad(Wr1, ((0, 0), (0, 20)))
    att1p = jnp.pad(att1, (0, 20))
    b1p = jnp.pad(b1, (0, 20)).reshape(1, 2 * H1)

    xl0, xl1h, xr0, xr1h = _t1(x, wl1p[:, :H1], wl1p[:, H1:],
                               wr1p[:, :H1], wr1p[:, H1:])
    ex1, sp1 = _gat_stage1(src_p, dst_p, att1p, [(xl0, xr0), (xl1h, xr1h)], H1)
    o1a = _gat_stage2(src_p, dst_p, ex1, sp1, xl0, H1)
    o1b = _gat_stage2(src_p, dst_p, ex1, sp1, xl1h, H1)

    wl2p = jnp.pad(Wl2, ((0, 20), (0, 12)))
    wr2p = jnp.pad(Wr2, ((0, 20), (0, 12)))
    att2p = jnp.pad(att2, (0, 12))
    b2p = jnp.pad(b2, (0, 12)).reshape(1, H2)
    xl2, xr2 = _t2(o1a, o1b, b1p, wl2p, wr2p)
    ex2, sp2 = _gat_stage1(src_p, dst_p, att2p, [(xl2, xr2)], H2)
    o2 = _gat_stage2(src_p, dst_p, ex2, sp2, xl2, H2)

    wlin1p = jnp.pad(Wlin1, ((0, 0), (0, 12)))
    blin1p = jnp.pad(blin1, (0, 12)).reshape(1, H2)
    wlin2p = jnp.pad(Wlin2, ((0, 0), (0, 12)))
    blin2p = jnp.pad(blin2, (0, 12)).reshape(1, H2)
    w3p = jnp.pad(jnp.concatenate([Wl3, Wr3], axis=1), ((0, 12), (0, 126)))
    z, o3c = _t3(o2, b2p, x, wlin1p, blin1p, wlin2p, blin2p, w3p)

    xl3 = jnp.pad(o3c[:, 0], (0, NPAD - N))
    xr3 = jnp.pad(o3c[:, 1], (0, NPAD - N))
    att3b = jnp.broadcast_to(att3, (16,))
    ex3, sp3 = _l3_stage1(src_p, dst_p, xl3, xr3, att3b)
    p3 = _l3_stage2(src_p, dst_p, ex3, sp3, xl3)

    pa = jnp.pad(edge_index[0], (0, ELP - E))
    pb = jnp.pad(edge_index[1], (0, ELP - E))
    na = jnp.pad(neg_edge_index[0], (0, ELP - E))
    nb = jnp.pad(neg_edge_index[1], (0, ELP - E))
    dp, dn = _loss_dots(pa, pb, na, nb, z)
    dp2 = dp[:E].reshape(1250, 128)
    dn2 = dn[:E].reshape(1250, 128)

    rl, o3 = _t4(dp2, dn2, p3, b3.reshape(1, 1))
    out = o3[:N].reshape(N, 1)
    r_loss = rl[0, 0]
    return (out, r_loss, c1, c2)


# speedup vs baseline: 2.5174x; 2.5174x over previous
import jax
import jax.numpy as jnp
from jax import lax
from jax.experimental import pallas as pl
from jax.experimental.pallas import tpu as pltpu
from jax.experimental.pallas import tpu_sc as plsc

F32 = jnp.float32
I32 = jnp.int32

N = 10000
NPAD = 10240
NSLICE = NPAD // 16
E = 160000
ET = E + N
ETP = 170496
EW = ETP // 32
NCH = EW // 16
ELP = 160256
EWL = ELP // 32
NCHL = EWL // 16

DW = 128
NJ = DW // 16


def _mesh():
    return plsc.VectorSubcoreMesh(core_axis_name="c", subcore_axis_name="s")


def _sc_params():
    return pltpu.CompilerParams(needs_layout_passes=False)


def _worker_id():
    return lax.axis_index("c") * 16 + lax.axis_index("s")


def _zero_1d(ref, nwords):
    zero = jnp.zeros((16,), F32)

    def zb(k, carry):
        ref[pl.ds(k * 16, 16)] = zero
        return carry

    lax.fori_loop(0, nwords // 16, zb, None)


def _combine_to_hbm(local_ref, shs, tmp_v, red_v, out_hbm):
    c = lax.axis_index("c")
    sb = lax.axis_index("s")
    pltpu.sync_copy(local_ref, shs.at[sb])
    plsc.subcore_barrier()
    off = sb * NSLICE
    pltpu.sync_copy(shs.at[0, pl.ds(off, NSLICE)], red_v)
    for r in range(1, 16):
        pltpu.sync_copy(shs.at[r, pl.ds(off, NSLICE)], tmp_v)

        def addk(k, carry):
            red_v[pl.ds(k * 16, 16)] = (
                red_v[pl.ds(k * 16, 16)] + tmp_v[pl.ds(k * 16, 16)]
            )
            return carry

        lax.fori_loop(0, NSLICE // 16, addk, None)
    pltpu.sync_copy(red_v, out_hbm.at[c, pl.ds(off, NSLICE)])


def _gat_stage1(src_p, dst_p, att_p, pairs):
    npairs = len(pairs)
    NH = npairs * DW

    def body(src_hbm, dst_hbm, att_hbm, *rest):
        tabs = rest[:2 * npairs]
        ex_hbm, spart_hbm = rest[2 * npairs], rest[2 * npairs + 1]
        scr = rest[2 * npairs + 2:]
        att_v, src_v, dst_v = scr[0], scr[1], scr[2]
        rows = scr[3:3 + 2 * npairs]
        P, ex_v, s_loc, tmp_v, red_v, shs, sem = scr[3 + 2 * npairs:]

        wid = _worker_id()
        pltpu.sync_copy(att_hbm, att_v)
        _zero_1d(s_loc, NPAD)
        att_regs = [att_v[pl.ds(k * 16, 16)] for k in range(NH // 16)]
        iot = lax.iota(I32, 16)

        def chunk(i, carry):
            base = wid * EW + i * 16
            pltpu.sync_copy(src_hbm.at[pl.ds(base, 16)], src_v)
            pltpu.sync_copy(dst_hbm.at[pl.ds(base, 16)], dst_v)
            cps = []
            for p in range(npairs):
                cps.append(
                    pltpu.async_copy(tabs[2 * p].at[src_v], rows[2 * p], sem))
                cps.append(
                    pltpu.async_copy(tabs[2 * p + 1].at[dst_v], rows[2 * p + 1],
                                     sem))
            for cp in cps:
                cp.wait()

            def edge(e, ecarry):
                acc = jnp.zeros((16,), F32)
                for p in range(npairs):
                    for j in range(NJ):
                        v = (rows[2 * p][e, pl.ds(j * 16, 16)]
                             + rows[2 * p + 1][e, pl.ds(j * 16, 16)])
                        v = jnp.where(v >= 0, v, 0.2 * v)
                        acc = acc + v * att_regs[p * NJ + j]
                P[pl.ds(e * 16, 16)] = acc
                return ecarry

            lax.fori_loop(0, 16, edge, None)
            t = jnp.zeros((16,), F32)
            for col in range(16):
                t = t + plsc.load_gather(P, [iot * 16 + col])
            mask = (base + iot) < ET
            exv = jnp.where(mask, jnp.exp(t), 0.0)
            ex_v[...] = exv
            pltpu.sync_copy(ex_v, ex_hbm.at[pl.ds(base, 16)])
            plsc.addupdate_scatter(s_loc, [dst_v[...]], exv)
            return carry

        lax.fori_loop(0, NCH, chunk, None)
        _combine_to_hbm(s_loc, shs, tmp_v, red_v, spart_hbm)

    scratch = (
        [pltpu.VMEM((NH,), F32), pltpu.VMEM((16,), I32), pltpu.VMEM((16,), I32)]
        + [pltpu.VMEM((16, DW), F32)] * (2 * npairs)
        + [pltpu.VMEM((256,), F32), pltpu.VMEM((16,), F32),
           pltpu.VMEM((NPAD,), F32), pltpu.VMEM((NSLICE,), F32),
           pltpu.VMEM((NSLICE,), F32),
           pltpu.VMEM_SHARED((16, NPAD), F32), pltpu.SemaphoreType.DMA]
    )
    flat_tabs = [a for pair in pairs for a in pair]
    fn = pl.kernel(
        body,
        out_type=[jax.ShapeDtypeStruct((ETP,), F32),
                  jax.ShapeDtypeStruct((2, NPAD), F32)],
        mesh=_mesh(),
        compiler_params=_sc_params(),
        scratch_types=scratch,
    )
    return fn(src_p, dst_p, att_p, *flat_tabs)


def _gat_stage2(src_p, dst_p, ex, spart, xl):
    def body(src_hbm, dst_hbm, ex_hbm, sp_hbm, xl_hbm, op_hbm,
             s_tot, tmp_big, src_v, dst_v, ex_v, rows, a_buf, zrow, osh, sem):
        c = lax.axis_index("c")
        sb = lax.axis_index("s")
        wid = _worker_id()
        pltpu.sync_copy(sp_hbm.at[0], s_tot)
        pltpu.sync_copy(sp_hbm.at[1], tmp_big)

        def sk(k, carry):
            s_tot[pl.ds(k * 16, 16)] = (
                s_tot[pl.ds(k * 16, 16)] + tmp_big[pl.ds(k * 16, 16)] + 1e-16
            )
            return carry

        lax.fori_loop(0, NPAD // 16, sk, None)

        zero = jnp.zeros((16,), F32)
        for r in range(16):
            for j in range(NJ):
                zrow[r, pl.ds(j * 16, 16)] = zero
        for r in range(NSLICE // 16):
            pltpu.sync_copy(zrow, osh.at[pl.ds(sb * NSLICE + r * 16, 16)])
        plsc.subcore_barrier()

        def chunk(i, carry):
            base = wid * EW + i * 16
            pltpu.sync_copy(src_hbm.at[pl.ds(base, 16)], src_v)
            pltpu.sync_copy(dst_hbm.at[pl.ds(base, 16)], dst_v)
            pltpu.sync_copy(ex_hbm.at[pl.ds(base, 16)], ex_v)
            pltpu.async_copy(xl_hbm.at[src_v], rows, sem).wait()
            sv = plsc.load_gather(s_tot, [dst_v[...]])
            a_buf[...] = ex_v[...] / sv

            def edge(e, ecarry):
                av = plsc.load_gather(a_buf, [jnp.full((16,), 0, I32) + e])
                for j in range(NJ):
                    rows[e, pl.ds(j * 16, 16)] = rows[e, pl.ds(j * 16, 16)] * av
                return ecarry

            lax.fori_loop(0, 16, edge, None)
            pltpu.sync_copy(rows, osh.at[dst_v], add=True)
            return carry

        lax.fori_loop(0, NCH, chunk, None)
        plsc.subcore_barrier()
        pltpu.sync_copy(osh.at[pl.ds(sb * NSLICE, NSLICE)],
                        op_hbm.at[c, pl.ds(sb * NSLICE, NSLICE)])

    scratch = [
        pltpu.VMEM((NPAD,), F32), pltpu.VMEM((NPAD,), F32),
        pltpu.VMEM((16,), I32), pltpu.VMEM((16,), I32), pltpu.VMEM((16,), F32),
        pltpu.VMEM((16, DW), F32), pltpu.VMEM((16,), F32),
        pltpu.VMEM((16, DW), F32),
        pltpu.VMEM_SHARED((NPAD, DW), F32), pltpu.SemaphoreType.DMA,
    ]
    fn = pl.kernel(
        body,
        out_type=jax.ShapeDtypeStruct((2, NPAD, DW), F32),
        mesh=_mesh(),
        compiler_params=_sc_params(),
        scratch_types=scratch,
    )
    return fn(src_p, dst_p, ex, spart, xl)


def _l3_stage1(src_p, dst_p, xl3, xr3, att3b):
    def body(src_hbm, dst_hbm, xl_hbm, xr_hbm, att_hbm, ex_hbm, spart_hbm,
             xl_v, xr_v, att_v, src_v, dst_v, ex_v, s_loc, tmp_v, red_v, shs,
             sem):
        wid = _worker_id()
        pltpu.sync_copy(xl_hbm, xl_v)
        pltpu.sync_copy(xr_hbm, xr_v)
        pltpu.sync_copy(att_hbm, att_v)
        _zero_1d(s_loc, NPAD)
        att_reg = att_v[...]
        iot = lax.iota(I32, 16)

        def chunk(i, carry):
            base = wid * EW + i * 16
            pltpu.sync_copy(src_hbm.at[pl.ds(base, 16)], src_v)
            pltpu.sync_copy(dst_hbm.at[pl.ds(base, 16)], dst_v)
            sreg = src_v[...]
            dreg = dst_v[...]
            v = plsc.load_gather(xl_v, [sreg]) + plsc.load_gather(xr_v, [dreg])
            t = jnp.where(v >= 0, v, 0.2 * v) * att_reg
            mask = (base + iot) < ET
            exv = jnp.where(mask, jnp.exp(t), 0.0)
            ex_v[...] = exv
            pltpu.sync_copy(ex_v, ex_hbm.at[pl.ds(base, 16)])
            plsc.addupdate_scatter(s_loc, [dreg], exv)
            return carry

        lax.fori_loop(0, NCH, chunk, None)
        _combine_to_hbm(s_loc, shs, tmp_v, red_v, spart_hbm)

    scratch = [
        pltpu.VMEM((NPAD,), F32), pltpu.VMEM((NPAD,), F32),
        pltpu.VMEM((16,), F32),
        pltpu.VMEM((16,), I32), pltpu.VMEM((16,), I32), pltpu.VMEM((16,), F32),
        pltpu.VMEM((NPAD,), F32), pltpu.VMEM((NSLICE,), F32),
        pltpu.VMEM((NSLICE,), F32),
        pltpu.VMEM_SHARED((16, NPAD), F32), pltpu.SemaphoreType.DMA,
    ]
    fn = pl.kernel(
        body,
        out_type=[jax.ShapeDtypeStruct((ETP,), F32),
                  jax.ShapeDtypeStruct((2, NPAD), F32)],
        mesh=_mesh(),
        compiler_params=_sc_params(),
        scratch_types=scratch,
    )
    return fn(src_p, dst_p, xl3, xr3, att3b)


def _l3_stage2(src_p, dst_p, ex3, spart, xl3):
    def body(src_hbm, dst_hbm, ex_hbm, sp_hbm, xl_hbm, op_hbm,
             xl_v, s_tot, tmp_big, src_v, dst_v, ex_v, o_loc, tmp_v, red_v,
             shs, sem):
        wid = _worker_id()
        pltpu.sync_copy(xl_hbm, xl_v)
        pltpu.sync_copy(sp_hbm.at[0], s_tot)
        pltpu.sync_copy(sp_hbm.at[1], tmp_big)

        def sk(k, carry):
            s_tot[pl.ds(k * 16, 16)] = (
                s_tot[pl.ds(k * 16, 16)] + tmp_big[pl.ds(k * 16, 16)] + 1e-16
            )
            return carry

        lax.fori_loop(0, NPAD // 16, sk, None)
        _zero_1d(o_loc, NPAD)

        def chunk(i, carry):
            base = wid * EW + i * 16
            pltpu.sync_copy(src_hbm.at[pl.ds(base, 16)], src_v)
            pltpu.sync_copy(dst_hbm.at[pl.ds(base, 16)], dst_v)
            pltpu.sync_copy(ex_hbm.at[pl.ds(base, 16)], ex_v)
            sreg = src_v[...]
            dreg = dst_v[...]
            alpha = ex_v[...] / plsc.load_gather(s_tot, [dreg])
            o = alpha * plsc.load_gather(xl_v, [sreg])
            plsc.addupdate_scatter(o_loc, [dreg], o)
            return carry

        lax.fori_loop(0, NCH, chunk, None)
        _combine_to_hbm(o_loc, shs, tmp_v, red_v, op_hbm)

    scratch = [
        pltpu.VMEM((NPAD,), F32), pltpu.VMEM((NPAD,), F32),
        pltpu.VMEM((NPAD,), F32),
        pltpu.VMEM((16,), I32), pltpu.VMEM((16,), I32), pltpu.VMEM((16,), F32),
        pltpu.VMEM((NPAD,), F32), pltpu.VMEM((NSLICE,), F32),
        pltpu.VMEM((NSLICE,), F32),
        pltpu.VMEM_SHARED((16, NPAD), F32), pltpu.SemaphoreType.DMA,
    ]
    fn = pl.kernel(
        body,
        out_type=jax.ShapeDtypeStruct((2, NPAD), F32),
        mesh=_mesh(),
        compiler_params=_sc_params(),
        scratch_types=scratch,
    )
    return fn(src_p, dst_p, ex3, spart, xl3)


def _loss_dots(pa, pb, na, nb, z):
    def body(pa_hbm, pb_hbm, na_hbm, nb_hbm, z_hbm, dp_hbm, dn_hbm,
             ia, ib, za, zb, P, d_v, sem):
        wid = _worker_id()
        iot = lax.iota(I32, 16)
        for (a_hbm, b_hbm, o_hbm) in ((pa_hbm, pb_hbm, dp_hbm),
                                      (na_hbm, nb_hbm, dn_hbm)):
            def chunk(i, carry, a_hbm=a_hbm, b_hbm=b_hbm, o_hbm=o_hbm):
                base = wid * EWL + i * 16
                pltpu.sync_copy(a_hbm.at[pl.ds(base, 16)], ia)
                pltpu.sync_copy(b_hbm.at[pl.ds(base, 16)], ib)
                cp1 = pltpu.async_copy(z_hbm.at[ia], za, sem)
                cp2 = pltpu.async_copy(z_hbm.at[ib], zb, sem)
                cp1.wait()
                cp2.wait()

                def edge(e, ecarry):
                    acc = jnp.zeros((16,), F32)
                    for j in range(NJ):
                        acc = (acc
                               + za[e, pl.ds(j * 16, 16)]
                               * zb[e, pl.ds(j * 16, 16)])
                    P[pl.ds(e * 16, 16)] = acc
                    return ecarry

                lax.fori_loop(0, 16, edge, None)
                t = jnp.zeros((16,), F32)
                for col in range(16):
                    t = t + plsc.load_gather(P, [iot * 16 + col])
                d_v[...] = t
                pltpu.sync_copy(d_v, o_hbm.at[pl.ds(base, 16)])
                return carry

            lax.fori_loop(0, NCHL, chunk, None)

    scratch = [
        pltpu.VMEM((16,), I32), pltpu.VMEM((16,), I32),
        pltpu.VMEM((16, DW), F32), pltpu.VMEM((16, DW), F32),
        pltpu.VMEM((256,), F32), pltpu.VMEM((16,), F32),
        pltpu.SemaphoreType.DMA,
    ]
    fn = pl.kernel(
        body,
        out_type=[jax.ShapeDtypeStruct((ELP,), F32),
                  jax.ShapeDtypeStruct((ELP,), F32)],
        mesh=_mesh(),
        compiler_params=_sc_params(),
        scratch_types=scratch,
    )
    return fn(pa, pb, na, nb, z)


_RB = 2000


def _t1(x, ws):
    nw = len(ws)

    def body(*refs):
        x_ref = refs[0]
        w_refs = refs[1:1 + nw]
        o_refs = refs[1 + nw:]
        xb = x_ref[...]
        for w, o in zip(w_refs, o_refs):
            o[...] = jnp.dot(xb, w[...], preferred_element_type=F32)

    return pl.pallas_call(
        body,
        grid=(N // _RB,),
        in_specs=[pl.BlockSpec((_RB, 128), lambda i: (i, 0))]
        + [pl.BlockSpec((128, DW), lambda i: (0, 0))] * nw,
        out_specs=[pl.BlockSpec((_RB, DW), lambda i: (i, 0))] * nw,
        out_shape=[jax.ShapeDtypeStruct((N, DW), F32)] * nw,
    )(x, *ws)


def _t2(o1parts, b1p, wl2p, wr2p):
    def body(oa_ref, ob_ref, oc_ref, b1_ref, wl_ref, wr_ref, xl2_ref, xr2_ref):
        h = jnp.concatenate(
            [oa_ref[0] + oa_ref[1], ob_ref[0] + ob_ref[1],
             oc_ref[0] + oc_ref[1]], axis=1)
        h = jnp.maximum(h + b1_ref[...], 0.0)
        xl2_ref[...] = jnp.dot(h, wl_ref[...], preferred_element_type=F32)
        xr2_ref[...] = jnp.dot(h, wr_ref[...], preferred_element_type=F32)

    return pl.pallas_call(
        body,
        grid=(N // _RB,),
        in_specs=[pl.BlockSpec((2, _RB, DW), lambda i: (0, i, 0))] * 3
        + [
            pl.BlockSpec((1, 3 * DW), lambda i: (0, 0)),
            pl.BlockSpec((3 * DW, DW), lambda i: (0, 0)),
            pl.BlockSpec((3 * DW, DW), lambda i: (0, 0)),
        ],
        out_specs=[pl.BlockSpec((_RB, DW), lambda i: (i, 0))] * 2,
        out_shape=[jax.ShapeDtypeStruct((N, DW), F32)] * 2,
    )(*o1parts, b1p, wl2p, wr2p)


def _t3(o2, b2p, x, wlin1p, blin1p, wlin2p, blin2p, w3p):
    def body(o2_ref, b2_ref, x_ref, w1_ref, bl1_ref, w2_ref, bl2_ref, w3_ref,
             z_ref, o3c_ref):
        x1 = jnp.maximum(o2_ref[0] + o2_ref[1] + b2_ref[...], 0.0)
        xb = x_ref[...]
        t1 = jnp.maximum(
            jnp.dot(xb, w1_ref[...], preferred_element_type=F32)
            + bl1_ref[...], 0.0)
        xs = x1 + t1
        t2 = jnp.maximum(
            jnp.dot(xb, w2_ref[...], preferred_element_type=F32)
            + bl2_ref[...], 0.0)
        z_ref[...] = x1 + t2
        o3c_ref[...] = jnp.dot(xs, w3_ref[...], preferred_element_type=F32)

    return pl.pallas_call(
        body,
        grid=(N // _RB,),
        in_specs=[
            pl.BlockSpec((2, _RB, DW), lambda i: (0, i, 0)),
            pl.BlockSpec((1, DW), lambda i: (0, 0)),
            pl.BlockSpec((_RB, 128), lambda i: (i, 0)),
            pl.BlockSpec((128, DW), lambda i: (0, 0)),
            pl.BlockSpec((1, DW), lambda i: (0, 0)),
            pl.BlockSpec((128, DW), lambda i: (0, 0)),
            pl.BlockSpec((1, DW), lambda i: (0, 0)),
            pl.BlockSpec((DW, 128), lambda i: (0, 0)),
        ],
        out_specs=[
            pl.BlockSpec((_RB, DW), lambda i: (i, 0)),
            pl.BlockSpec((_RB, 128), lambda i: (i, 0)),
        ],
        out_shape=[
            jax.ShapeDtypeStruct((N, DW), F32),
            jax.ShapeDtypeStruct((N, 128), F32),
        ],
    )(o2, b2p, x, wlin1p, blin1p, wlin2p, blin2p, w3p)


def _t4(dp2, dn2, p3, b3r):
    def body(dp_ref, dn_ref, p3_ref, b3_ref, rl_ref, o3_ref):
        p = jax.nn.sigmoid(dp_ref[...])
        pls = -jnp.mean(jnp.log(p + 1e-15))
        q = jax.nn.sigmoid(dn_ref[...])
        nls = -jnp.mean(jnp.log(1.0 - q + 1e-15))
        rl_ref[...] = jnp.reshape(pls + nls, (1, 1))
        o3_ref[...] = p3_ref[0] + p3_ref[1] + b3_ref[...]

    return pl.pallas_call(
        body,
        out_shape=[jax.ShapeDtypeStruct((1, 1), F32),
                   jax.ShapeDtypeStruct((NPAD,), F32)],
    )(dp2, dn2, p3, b3r)


def kernel(x, edge_index, neg_edge_index, Wl1, Wr1, att1, b1, Wl2, Wr2, att2,
           b2, Wl3, Wr3, att3, b3, Wlin1, blin1, Wlin2, blin2, c1, c2):
    loop = jnp.arange(N, dtype=edge_index.dtype)
    src = jnp.concatenate([edge_index[0], loop])
    dst = jnp.concatenate([edge_index[1], loop])
    src_p = jnp.pad(src, (0, ETP - ET))
    dst_p = jnp.pad(dst, (0, ETP - ET))

    wl1p = jnp.pad(Wl1, ((0, 0), (0, 84)))
    wr1p = jnp.pad(Wr1, ((0, 0), (0, 84)))
    att1p = jnp.pad(att1, (0, 84))
    b1p = jnp.pad(b1, (0, 84)).reshape(1, 3 * DW)

    t1outs = _t1(x, [wl1p[:, :DW], wl1p[:, DW:2 * DW], wl1p[:, 2 * DW:],
                     wr1p[:, :DW], wr1p[:, DW:2 * DW], wr1p[:, 2 * DW:]])
    xl1s, xr1s = t1outs[:3], t1outs[3:]
    pairs1 = list(zip(xl1s, xr1s))
    ex1, sp1 = _gat_stage1(src_p, dst_p, att1p, pairs1)
    o1parts = [_gat_stage2(src_p, dst_p, ex1, sp1, t) for t in xl1s]

    wl2p = jnp.pad(Wl2, ((0, 284), (0, 28)))
    wr2p = jnp.pad(Wr2, ((0, 284), (0, 28)))
    att2p = jnp.pad(att2, (0, 28))
    b2p = jnp.pad(b2, (0, 28)).reshape(1, DW)
    xl2, xr2 = _t2(o1parts, b1p, wl2p, wr2p)
    ex2, sp2 = _gat_stage1(src_p, dst_p, att2p, [(xl2, xr2)])
    o2 = _gat_stage2(src_p, dst_p, ex2, sp2, xl2)

    wlin1p = jnp.pad(Wlin1, ((0, 0), (0, 28)))
    blin1p = jnp.pad(blin1, (0, 28)).reshape(1, DW)
    wlin2p = jnp.pad(Wlin2, ((0, 0), (0, 28)))
    blin2p = jnp.pad(blin2, (0, 28)).reshape(1, DW)
    w3p = jnp.pad(jnp.concatenate([Wl3, Wr3], axis=1), ((0, 28), (0, 126)))
    z, o3c = _t3(o2, b2p, x, wlin1p, blin1p, wlin2p, blin2p, w3p)

    xl3 = jnp.pad(o3c[:, 0], (0, NPAD - N))
    xr3 = jnp.pad(o3c[:, 1], (0, NPAD - N))
    att3b = jnp.broadcast_to(att3, (16,))
    ex3, sp3 = _l3_stage1(src_p, dst_p, xl3, xr3, att3b)
    p3 = _l3_stage2(src_p, dst_p, ex3, sp3, xl3)

    pa = jnp.pad(edge_index[0], (0, ELP - E))
    pb = jnp.pad(edge_index[1], (0, ELP - E))
    na = jnp.pad(neg_edge_index[0], (0, ELP - E))
    nb = jnp.pad(neg_edge_index[1], (0, ELP - E))
    dp, dn = _loss_dots(pa, pb, na, nb, z)
    dp2 = dp[:E].reshape(1250, 128)
    dn2 = dn[:E].reshape(1250, 128)

    rl, o3 = _t4(dp2, dn2, p3, b3)
    out = o3[:N].reshape(N, 1)
    r_loss = rl[0, 0]
    return (out, r_loss, c1, c2)


# speedup vs baseline: 8.2941x; 3.2948x over previous
import jax
import jax.numpy as jnp
from jax import lax
from jax.experimental import pallas as pl
from jax.experimental.pallas import tpu as pltpu
from jax.experimental.pallas import tpu_sc as plsc

F32 = jnp.float32
I32 = jnp.int32

N = 10000
NPAD = 10240
NSLICE = NPAD // 16
E = 160000
ET = E + N
ETP = 171008
EW = ETP // 32
NCH = EW // 16
ELP = 160768
EWL = ELP // 32
NCHL = EWL // 16

DW = 128
NJ = DW // 16


def _mesh():
    return plsc.VectorSubcoreMesh(core_axis_name="c", subcore_axis_name="s")


def _sc_params():
    return pltpu.CompilerParams(needs_layout_passes=False)


def _worker_id():
    return lax.axis_index("c") * 16 + lax.axis_index("s")


def _zero_1d(ref, nwords):
    zero = jnp.zeros((16,), F32)

    def zb(k, carry):
        ref[pl.ds(k * 16, 16)] = zero
        return carry

    lax.fori_loop(0, nwords // 16, zb, None)


def _combine_to_hbm(local_ref, shs, tmp_v, red_v, out_hbm):
    c = lax.axis_index("c")
    sb = lax.axis_index("s")
    pltpu.sync_copy(local_ref, shs.at[sb])
    plsc.subcore_barrier()
    off = sb * NSLICE
    pltpu.sync_copy(shs.at[0, pl.ds(off, NSLICE)], red_v)
    for r in range(1, 16):
        pltpu.sync_copy(shs.at[r, pl.ds(off, NSLICE)], tmp_v)

        def addk(k, carry):
            red_v[pl.ds(k * 16, 16)] = (
                red_v[pl.ds(k * 16, 16)] + tmp_v[pl.ds(k * 16, 16)]
            )
            return carry

        lax.fori_loop(0, NSLICE // 16, addk, None)
    pltpu.sync_copy(red_v, out_hbm.at[c, pl.ds(off, NSLICE)])


def _load_s_tot(sp_hbm, s_tot, tmp_big):
    pltpu.sync_copy(sp_hbm.at[0], s_tot)
    pltpu.sync_copy(sp_hbm.at[1], tmp_big)

    def sk(k, carry):
        s_tot[pl.ds(k * 16, 16)] = (
            s_tot[pl.ds(k * 16, 16)] + tmp_big[pl.ds(k * 16, 16)] + 1e-16
        )
        return carry

    lax.fori_loop(0, NPAD // 16, sk, None)


def _gat_stage1(src_p, dst_p, att_p, pairs):
    npairs = len(pairs)
    nt = 2 * npairs
    NH = npairs * DW

    def body(src_hbm, dst_hbm, att_hbm, *rest):
        tabs = rest[:nt]
        ex_hbm, spart_hbm = rest[nt], rest[nt + 1]
        scr = rest[nt + 2:]
        att_v, src_big, dst_big, ex_big = scr[0], scr[1], scr[2], scr[3]
        rows0 = scr[4:4 + nt]
        rows1 = scr[4 + nt:4 + 2 * nt]
        rows = (rows0, rows1)
        P, s_loc, tmp_v, red_v, shs, sem0, sem1 = scr[4 + 2 * nt:]
        sems = (sem0, sem1)

        wid = _worker_id()
        wbase = wid * EW
        pltpu.sync_copy(att_hbm, att_v)
        pltpu.sync_copy(src_hbm.at[pl.ds(wbase, EW)], src_big)
        pltpu.sync_copy(dst_hbm.at[pl.ds(wbase, EW)], dst_big)
        _zero_1d(s_loc, NPAD)
        att_regs = [att_v[pl.ds(k * 16, 16)] for k in range(NH // 16)]
        iot = lax.iota(I32, 16)

        def issue(k, b):
            sreg = src_big[pl.ds(k * 16, 16)]
            dreg = dst_big[pl.ds(k * 16, 16)]
            for p in range(npairs):
                pltpu.async_copy(tabs[2 * p].at[sreg], rows[b][2 * p], sems[b])
                pltpu.async_copy(tabs[2 * p + 1].at[dreg], rows[b][2 * p + 1],
                                 sems[b])

        def wait(b):
            for p in range(nt):
                pltpu.make_async_copy(tabs[p].at[iot], rows[b][p],
                                      sems[b]).wait()

        issue(0, 0)
        issue(1, 1)

        def group(g, carry):
            for b in (0, 1):
                k = g * 2 + b
                wait(b)

                def edge(e, ecarry):
                    acc0 = jnp.zeros((16,), F32)
                    acc1 = jnp.zeros((16,), F32)
                    for p in range(npairs):
                        for j in range(NJ):
                            v = (rows[b][2 * p][e, pl.ds(j * 16, 16)]
                                 + rows[b][2 * p + 1][e, pl.ds(j * 16, 16)])
                            v = jnp.where(v >= 0, v, 0.2 * v)
                            if j % 2 == 0:
                                acc0 = acc0 + v * att_regs[p * NJ + j]
                            else:
                                acc1 = acc1 + v * att_regs[p * NJ + j]
                    P[pl.ds(e * 16, 16)] = acc0 + acc1
                    return ecarry

                lax.fori_loop(0, 16, edge, None)
                t0 = jnp.zeros((16,), F32)
                t1 = jnp.zeros((16,), F32)
                for col in range(0, 16, 2):
                    t0 = t0 + plsc.load_gather(P, [iot * 16 + col])
                    t1 = t1 + plsc.load_gather(P, [iot * 16 + (col + 1)])
                base = wbase + k * 16
                mask = (base + iot) < ET
                exv = jnp.where(mask, jnp.exp(t0 + t1), 0.0)
                ex_big[pl.ds(k * 16, 16)] = exv
                plsc.addupdate_scatter(s_loc, [dst_big[pl.ds(k * 16, 16)]], exv)

                @pl.when(k + 2 < NCH)
                def _():
                    issue(k + 2, b)
            return carry

        lax.fori_loop(0, NCH // 2, group, None)
        pltpu.sync_copy(ex_big, ex_hbm.at[pl.ds(wbase, EW)])
        _combine_to_hbm(s_loc, shs, tmp_v, red_v, spart_hbm)

    scratch = (
        [pltpu.VMEM((NH,), F32), pltpu.VMEM((EW,), I32), pltpu.VMEM((EW,), I32),
         pltpu.VMEM((EW,), F32)]
        + [pltpu.VMEM((16, DW), F32)] * (2 * nt)
        + [pltpu.VMEM((256,), F32),
           pltpu.VMEM((NPAD,), F32), pltpu.VMEM((NSLICE,), F32),
           pltpu.VMEM((NSLICE,), F32),
           pltpu.VMEM_SHARED((16, NPAD), F32),
           pltpu.SemaphoreType.DMA, pltpu.SemaphoreType.DMA]
    )
    flat_tabs = [a for pair in pairs for a in pair]
    fn = pl.kernel(
        body,
        out_type=[jax.ShapeDtypeStruct((ETP,), F32),
                  jax.ShapeDtypeStruct((2, NPAD), F32)],
        mesh=_mesh(),
        compiler_params=_sc_params(),
        scratch_types=scratch,
    )
    return fn(src_p, dst_p, att_p, *flat_tabs)


def _gat_stage2(src_p, dst_p, ex, spart, xl):
    def body(src_hbm, dst_hbm, ex_hbm, sp_hbm, xl_hbm, op_hbm,
             s_tot, tmp_big, src_big, dst_big, ex_big,
             rows0, rows1, a_buf, zrow, osh, sem0, sem1):
        c = lax.axis_index("c")
        sb = lax.axis_index("s")
        wid = _worker_id()
        wbase = wid * EW
        rows = (rows0, rows1)
        sems = (sem0, sem1)
        _load_s_tot(sp_hbm, s_tot, tmp_big)
        pltpu.sync_copy(src_hbm.at[pl.ds(wbase, EW)], src_big)
        pltpu.sync_copy(dst_hbm.at[pl.ds(wbase, EW)], dst_big)
        pltpu.sync_copy(ex_hbm.at[pl.ds(wbase, EW)], ex_big)
        iot = lax.iota(I32, 16)

        zero = jnp.zeros((16,), F32)
        for r in range(16):
            for j in range(NJ):
                zrow[r, pl.ds(j * 16, 16)] = zero
        for r in range(NSLICE // 16):
            pltpu.sync_copy(zrow, osh.at[pl.ds(sb * NSLICE + r * 16, 16)])
        plsc.subcore_barrier()

        def issue(k, b):
            pltpu.async_copy(xl_hbm.at[src_big[pl.ds(k * 16, 16)]], rows[b],
                             sems[b])

        issue(0, 0)
        issue(1, 1)

        def group(g, carry):
            for b in (0, 1):
                k = g * 2 + b
                pltpu.make_async_copy(xl_hbm.at[iot], rows[b], sems[b]).wait()
                dreg = dst_big[pl.ds(k * 16, 16)]
                sv = plsc.load_gather(s_tot, [dreg])
                a_buf[...] = ex_big[pl.ds(k * 16, 16)] / sv

                def edge(e, ecarry):
                    av = plsc.load_gather(a_buf, [jnp.full((16,), 0, I32) + e])
                    for j in range(NJ):
                        rows[b][e, pl.ds(j * 16, 16)] = (
                            rows[b][e, pl.ds(j * 16, 16)] * av)
                    return ecarry

                lax.fori_loop(0, 16, edge, None)
                pltpu.sync_copy(rows[b], osh.at[dreg], add=True)

                @pl.when(k + 2 < NCH)
                def _():
                    issue(k + 2, b)
            return carry

        lax.fori_loop(0, NCH // 2, group, None)
        plsc.subcore_barrier()
        pltpu.sync_copy(osh.at[pl.ds(sb * NSLICE, NSLICE)],
                        op_hbm.at[c, pl.ds(sb * NSLICE, NSLICE)])

    scratch = [
        pltpu.VMEM((NPAD,), F32), pltpu.VMEM((NPAD,), F32),
        pltpu.VMEM((EW,), I32), pltpu.VMEM((EW,), I32), pltpu.VMEM((EW,), F32),
        pltpu.VMEM((16, DW), F32), pltpu.VMEM((16, DW), F32),
        pltpu.VMEM((16,), F32), pltpu.VMEM((16, DW), F32),
        pltpu.VMEM_SHARED((NPAD, DW), F32),
        pltpu.SemaphoreType.DMA, pltpu.SemaphoreType.DMA,
    ]
    fn = pl.kernel(
        body,
        out_type=jax.ShapeDtypeStruct((2, NPAD, DW), F32),
        mesh=_mesh(),
        compiler_params=_sc_params(),
        scratch_types=scratch,
    )
    return fn(src_p, dst_p, ex, spart, xl)


def _l3_stage1(src_p, dst_p, xl3, xr3, att3b):
    def body(src_hbm, dst_hbm, xl_hbm, xr_hbm, att_hbm, ex_hbm, spart_hbm,
             xl_v, xr_v, att_v, src_big, dst_big, ex_big, s_loc, tmp_v, red_v,
             shs, sem):
        wid = _worker_id()
        wbase = wid * EW
        pltpu.sync_copy(xl_hbm, xl_v)
        pltpu.sync_copy(xr_hbm, xr_v)
        pltpu.sync_copy(att_hbm, att_v)
        pltpu.sync_copy(src_hbm.at[pl.ds(wbase, EW)], src_big)
        pltpu.sync_copy(dst_hbm.at[pl.ds(wbase, EW)], dst_big)
        _zero_1d(s_loc, NPAD)
        att_reg = att_v[...]
        iot = lax.iota(I32, 16)

        def chunk(i, carry):
            sreg = src_big[pl.ds(i * 16, 16)]
            dreg = dst_big[pl.ds(i * 16, 16)]
            v = plsc.load_gather(xl_v, [sreg]) + plsc.load_gather(xr_v, [dreg])
            t = jnp.where(v >= 0, v, 0.2 * v) * att_reg
            mask = (wbase + i * 16 + iot) < ET
            exv = jnp.where(mask, jnp.exp(t), 0.0)
            ex_big[pl.ds(i * 16, 16)] = exv
            plsc.addupdate_scatter(s_loc, [dreg], exv)
            return carry

        lax.fori_loop(0, NCH, chunk, None)
        pltpu.sync_copy(ex_big, ex_hbm.at[pl.ds(wbase, EW)])
        _combine_to_hbm(s_loc, shs, tmp_v, red_v, spart_hbm)

    scratch = [
        pltpu.VMEM((NPAD,), F32), pltpu.VMEM((NPAD,), F32),
        pltpu.VMEM((16,), F32),
        pltpu.VMEM((EW,), I32), pltpu.VMEM((EW,), I32), pltpu.VMEM((EW,), F32),
        pltpu.VMEM((NPAD,), F32), pltpu.VMEM((NSLICE,), F32),
        pltpu.VMEM((NSLICE,), F32),
        pltpu.VMEM_SHARED((16, NPAD), F32), pltpu.SemaphoreType.DMA,
    ]
    fn = pl.kernel(
        body,
        out_type=[jax.ShapeDtypeStruct((ETP,), F32),
                  jax.ShapeDtypeStruct((2, NPAD), F32)],
        mesh=_mesh(),
        compiler_params=_sc_params(),
        scratch_types=scratch,
    )
    return fn(src_p, dst_p, xl3, xr3, att3b)


def _l3_stage2(src_p, dst_p, ex3, spart, xl3):
    def body(src_hbm, dst_hbm, ex_hbm, sp_hbm, xl_hbm, op_hbm,
             xl_v, s_tot, tmp_big, src_big, dst_big, ex_big, o_loc, tmp_v,
             red_v, shs, sem):
        wid = _worker_id()
        wbase = wid * EW
        pltpu.sync_copy(xl_hbm, xl_v)
        _load_s_tot(sp_hbm, s_tot, tmp_big)
        pltpu.sync_copy(src_hbm.at[pl.ds(wbase, EW)], src_big)
        pltpu.sync_copy(dst_hbm.at[pl.ds(wbase, EW)], dst_big)
        pltpu.sync_copy(ex_hbm.at[pl.ds(wbase, EW)], ex_big)
        _zero_1d(o_loc, NPAD)

        def chunk(i, carry):
            sreg = src_big[pl.ds(i * 16, 16)]
            dreg = dst_big[pl.ds(i * 16, 16)]
            alpha = ex_big[pl.ds(i * 16, 16)] / plsc.load_gather(s_tot, [dreg])
            o = alpha * plsc.load_gather(xl_v, [sreg])
            plsc.addupdate_scatter(o_loc, [dreg], o)
            return carry

        lax.fori_loop(0, NCH, chunk, None)
        _combine_to_hbm(o_loc, shs, tmp_v, red_v, op_hbm)

    scratch = [
        pltpu.VMEM((NPAD,), F32), pltpu.VMEM((NPAD,), F32),
        pltpu.VMEM((NPAD,), F32),
        pltpu.VMEM((EW,), I32), pltpu.VMEM((EW,), I32), pltpu.VMEM((EW,), F32),
        pltpu.VMEM((NPAD,), F32), pltpu.VMEM((NSLICE,), F32),
        pltpu.VMEM((NSLICE,), F32),
        pltpu.VMEM_SHARED((16, NPAD), F32), pltpu.SemaphoreType.DMA,
    ]
    fn = pl.kernel(
        body,
        out_type=jax.ShapeDtypeStruct((2, NPAD), F32),
        mesh=_mesh(),
        compiler_params=_sc_params(),
        scratch_types=scratch,
    )
    return fn(src_p, dst_p, ex3, spart, xl3)


def _loss_dots(pa, pb, na, nb, z):
    def body(pa_hbm, pb_hbm, na_hbm, nb_hbm, z_hbm, dp_hbm, dn_hbm,
             a_big, b_big, d_big, za0, zb0, za1, zb1, P, sem0, sem1):
        wid = _worker_id()
        wbase = wid * EWL
        za = (za0, za1)
        zb = (zb0, zb1)
        sems = (sem0, sem1)
        iot = lax.iota(I32, 16)
        for (a_hbm, b_hbm, o_hbm) in ((pa_hbm, pb_hbm, dp_hbm),
                                      (na_hbm, nb_hbm, dn_hbm)):
            pltpu.sync_copy(a_hbm.at[pl.ds(wbase, EWL)], a_big)
            pltpu.sync_copy(b_hbm.at[pl.ds(wbase, EWL)], b_big)

            def issue(k, b):
                pltpu.async_copy(z_hbm.at[a_big[pl.ds(k * 16, 16)]], za[b],
                                 sems[b])
                pltpu.async_copy(z_hbm.at[b_big[pl.ds(k * 16, 16)]], zb[b],
                                 sems[b])

            issue(0, 0)
            issue(1, 1)

            def group(g, carry):
                for b in (0, 1):
                    k = g * 2 + b
                    pltpu.make_async_copy(z_hbm.at[iot], za[b], sems[b]).wait()
                    pltpu.make_async_copy(z_hbm.at[iot], zb[b], sems[b]).wait()

                    def edge(e, ecarry):
                        acc0 = jnp.zeros((16,), F32)
                        acc1 = jnp.zeros((16,), F32)
                        for j in range(NJ):
                            t = (za[b][e, pl.ds(j * 16, 16)]
                                 * zb[b][e, pl.ds(j * 16, 16)])
                            if j % 2 == 0:
                                acc0 = acc0 + t
                            else:
                                acc1 = acc1 + t
                        P[pl.ds(e * 16, 16)] = acc0 + acc1
                        return ecarry

                    lax.fori_loop(0, 16, edge, None)
                    t0 = jnp.zeros((16,), F32)
                    t1 = jnp.zeros((16,), F32)
                    for col in range(0, 16, 2):
                        t0 = t0 + plsc.load_gather(P, [iot * 16 + col])
                        t1 = t1 + plsc.load_gather(P, [iot * 16 + (col + 1)])
                    d_big[pl.ds(k * 16, 16)] = t0 + t1

                    @pl.when(k + 2 < NCHL)
                    def _():
                        issue(k + 2, b)
                return carry

            lax.fori_loop(0, NCHL // 2, group, None)
            pltpu.sync_copy(d_big, o_hbm.at[pl.ds(wbase, EWL)])

    scratch = [
        pltpu.VMEM((EWL,), I32), pltpu.VMEM((EWL,), I32),
        pltpu.VMEM((EWL,), F32),
        pltpu.VMEM((16, DW), F32), pltpu.VMEM((16, DW), F32),
        pltpu.VMEM((16, DW), F32), pltpu.VMEM((16, DW), F32),
        pltpu.VMEM((256,), F32),
        pltpu.SemaphoreType.DMA, pltpu.SemaphoreType.DMA,
    ]
    fn = pl.kernel(
        body,
        out_type=[jax.ShapeDtypeStruct((ELP,), F32),
                  jax.ShapeDtypeStruct((ELP,), F32)],
        mesh=_mesh(),
        compiler_params=_sc_params(),
        scratch_types=scratch,
    )
    return fn(pa, pb, na, nb, z)


_RB = 2000


def _t1(x, ws):
    nw = len(ws)

    def body(*refs):
        x_ref = refs[0]
        w_refs = refs[1:1 + nw]
        o_refs = refs[1 + nw:]
        xb = x_ref[...]
        for w, o in zip(w_refs, o_refs):
            o[...] = jnp.dot(xb, w[...], preferred_element_type=F32)

    return pl.pallas_call(
        body,
        grid=(N // _RB,),
        in_specs=[pl.BlockSpec((_RB, 128), lambda i: (i, 0))]
        + [pl.BlockSpec((128, DW), lambda i: (0, 0))] * nw,
        out_specs=[pl.BlockSpec((_RB, DW), lambda i: (i, 0))] * nw,
        out_shape=[jax.ShapeDtypeStruct((N, DW), F32)] * nw,
    )(x, *ws)


def _t2(o1parts, b1p, wl2p, wr2p):
    def body(oa_ref, ob_ref, oc_ref, b1_ref, wl_ref, wr_ref, xl2_ref, xr2_ref):
        h = jnp.concatenate(
            [oa_ref[0] + oa_ref[1], ob_ref[0] + ob_ref[1],
             oc_ref[0] + oc_ref[1]], axis=1)
        h = jnp.maximum(h + b1_ref[...], 0.0)
        xl2_ref[...] = jnp.dot(h, wl_ref[...], preferred_element_type=F32)
        xr2_ref[...] = jnp.dot(h, wr_ref[...], preferred_element_type=F32)

    return pl.pallas_call(
        body,
        grid=(N // _RB,),
        in_specs=[pl.BlockSpec((2, _RB, DW), lambda i: (0, i, 0))] * 3
        + [
            pl.BlockSpec((1, 3 * DW), lambda i: (0, 0)),
            pl.BlockSpec((3 * DW, DW), lambda i: (0, 0)),
            pl.BlockSpec((3 * DW, DW), lambda i: (0, 0)),
        ],
        out_specs=[pl.BlockSpec((_RB, DW), lambda i: (i, 0))] * 2,
        out_shape=[jax.ShapeDtypeStruct((N, DW), F32)] * 2,
    )(*o1parts, b1p, wl2p, wr2p)


def _t3(o2, b2p, x, wlin1p, blin1p, wlin2p, blin2p, w3p):
    def body(o2_ref, b2_ref, x_ref, w1_ref, bl1_ref, w2_ref, bl2_ref, w3_ref,
             z_ref, o3c_ref):
        x1 = jnp.maximum(o2_ref[0] + o2_ref[1] + b2_ref[...], 0.0)
        xb = x_ref[...]
        t1 = jnp.maximum(
            jnp.dot(xb, w1_ref[...], preferred_element_type=F32)
            + bl1_ref[...], 0.0)
        xs = x1 + t1
        t2 = jnp.maximum(
            jnp.dot(xb, w2_ref[...], preferred_element_type=F32)
            + bl2_ref[...], 0.0)
        z_ref[...] = x1 + t2
        o3c_ref[...] = jnp.dot(xs, w3_ref[...], preferred_element_type=F32)

    return pl.pallas_call(
        body,
        grid=(N // _RB,),
        in_specs=[
            pl.BlockSpec((2, _RB, DW), lambda i: (0, i, 0)),
            pl.BlockSpec((1, DW), lambda i: (0, 0)),
            pl.BlockSpec((_RB, 128), lambda i: (i, 0)),
            pl.BlockSpec((128, DW), lambda i: (0, 0)),
            pl.BlockSpec((1, DW), lambda i: (0, 0)),
            pl.BlockSpec((128, DW), lambda i: (0, 0)),
            pl.BlockSpec((1, DW), lambda i: (0, 0)),
            pl.BlockSpec((DW, 128), lambda i: (0, 0)),
        ],
        out_specs=[
            pl.BlockSpec((_RB, DW), lambda i: (i, 0)),
            pl.BlockSpec((_RB, 128), lambda i: (i, 0)),
        ],
        out_shape=[
            jax.ShapeDtypeStruct((N, DW), F32),
            jax.ShapeDtypeStruct((N, 128), F32),
        ],
    )(o2, b2p, x, wlin1p, blin1p, wlin2p, blin2p, w3p)


def _t4(dp2, dn2, p3, b3r):
    def body(dp_ref, dn_ref, p3_ref, b3_ref, rl_ref, o3_ref):
        p = jax.nn.sigmoid(dp_ref[...])
        pls = -jnp.mean(jnp.log(p + 1e-15))
        q = jax.nn.sigmoid(dn_ref[...])
        nls = -jnp.mean(jnp.log(1.0 - q + 1e-15))
        rl_ref[...] = jnp.reshape(pls + nls, (1, 1))
        o3_ref[...] = p3_ref[0] + p3_ref[1] + b3_ref[...]

    return pl.pallas_call(
        body,
        out_shape=[jax.ShapeDtypeStruct((1, 1), F32),
                   jax.ShapeDtypeStruct((NPAD,), F32)],
    )(dp2, dn2, p3, b3r)


def kernel(x, edge_index, neg_edge_index, Wl1, Wr1, att1, b1, Wl2, Wr2, att2,
           b2, Wl3, Wr3, att3, b3, Wlin1, blin1, Wlin2, blin2, c1, c2):
    loop = jnp.arange(N, dtype=edge_index.dtype)
    src = jnp.concatenate([edge_index[0], loop])
    dst = jnp.concatenate([edge_index[1], loop])
    src_p = jnp.pad(src, (0, ETP - ET))
    dst_p = jnp.pad(dst, (0, ETP - ET))

    wl1p = jnp.pad(Wl1, ((0, 0), (0, 84)))
    wr1p = jnp.pad(Wr1, ((0, 0), (0, 84)))
    att1p = jnp.pad(att1, (0, 84))
    b1p = jnp.pad(b1, (0, 84)).reshape(1, 3 * DW)

    t1outs = _t1(x, [wl1p[:, :DW], wl1p[:, DW:2 * DW], wl1p[:, 2 * DW:],
                     wr1p[:, :DW], wr1p[:, DW:2 * DW], wr1p[:, 2 * DW:]])
    xl1s, xr1s = t1outs[:3], t1outs[3:]
    pairs1 = list(zip(xl1s, xr1s))
    ex1, sp1 = _gat_stage1(src_p, dst_p, att1p, pairs1)
    o1parts = [_gat_stage2(src_p, dst_p, ex1, sp1, t) for t in xl1s]

    wl2p = jnp.pad(Wl2, ((0, 284), (0, 28)))
    wr2p = jnp.pad(Wr2, ((0, 284), (0, 28)))
    att2p = jnp.pad(att2, (0, 28))
    b2p = jnp.pad(b2, (0, 28)).reshape(1, DW)
    xl2, xr2 = _t2(o1parts, b1p, wl2p, wr2p)
    ex2, sp2 = _gat_stage1(src_p, dst_p, att2p, [(xl2, xr2)])
    o2 = _gat_stage2(src_p, dst_p, ex2, sp2, xl2)

    wlin1p = jnp.pad(Wlin1, ((0, 0), (0, 28)))
    blin1p = jnp.pad(blin1, (0, 28)).reshape(1, DW)
    wlin2p = jnp.pad(Wlin2, ((0, 0), (0, 28)))
    blin2p = jnp.pad(blin2, (0, 28)).reshape(1, DW)
    w3p = jnp.pad(jnp.concatenate([Wl3, Wr3], axis=1), ((0, 28), (0, 126)))
    z, o3c = _t3(o2, b2p, x, wlin1p, blin1p, wlin2p, blin2p, w3p)

    xl3 = jnp.pad(o3c[:, 0], (0, NPAD - N))
    xr3 = jnp.pad(o3c[:, 1], (0, NPAD - N))
    att3b = jnp.broadcast_to(att3, (16,))
    ex3, sp3 = _l3_stage1(src_p, dst_p, xl3, xr3, att3b)
    p3 = _l3_stage2(src_p, dst_p, ex3, sp3, xl3)

    pa = jnp.pad(edge_index[0], (0, ELP - E))
    pb = jnp.pad(edge_index[1], (0, ELP - E))
    na = jnp.pad(neg_edge_index[0], (0, ELP - E))
    nb = jnp.pad(neg_edge_index[1], (0, ELP - E))
    dp, dn = _loss_dots(pa, pb, na, nb, z)
    dp2 = dp[:E].reshape(1250, 128)
    dn2 = dn[:E].reshape(1250, 128)

    rl, o3 = _t4(dp2, dn2, p3, b3)
    out = o3[:N].reshape(N, 1)
    r_loss = rl[0, 0]
    return (out, r_loss, c1, c2)
